# baseline (speedup 1.0000x reference)
import jax
import jax.numpy as jnp
from jax.experimental import pallas as pl
from jax.experimental.pallas import tpu as pltpu

P = 32
K = 1024
R = 4
EPS = 1e-6


def _split3(v):
    hi = v.astype(jnp.bfloat16)
    rem = v - hi.astype(jnp.float32)
    md = rem.astype(jnp.bfloat16)
    lo = (rem - md.astype(jnp.float32)).astype(jnp.bfloat16)
    return hi, md, lo


def _permute_exact(v, pm):
    hi, md, lo = _split3(v)
    dims = (((1,), (0,)), ((), ()))
    ph = jax.lax.dot_general(hi, pm, dims, preferred_element_type=jnp.float32)
    pmd = jax.lax.dot_general(md, pm, dims, preferred_element_type=jnp.float32)
    plo = jax.lax.dot_general(lo, pm, dims, preferred_element_type=jnp.float32)
    return (ph + pmd) + plo


def _rvq_body(tok_ref, wexp_ref, embed_ref,
              xq_ref, codes_ref, usage_ref, loss_ref,
              counts_ref, acc_ref, cb2_ref, cbpack_ref):
    i = pl.program_id(0)
    nsteps = pl.num_programs(0)
    D = tok_ref.shape[1]

    @pl.when(i == 0)
    def _init():
        counts_ref[...] = jnp.zeros_like(counts_ref)
        acc_ref[0] = 0.0
        acc_ref[1] = 0.0
        emb = embed_ref[...]
        cb2_ref[...] = jnp.sum(emb * emb, axis=2)
        hi, md, lo = _split3(emb)
        cbpack_ref[...] = jnp.concatenate([hi, md, lo], axis=2)

    jj = jax.lax.broadcasted_iota(jnp.int32, (D, D), 0)
    kk = jax.lax.broadcasted_iota(jnp.int32, (D, D), 1)
    pm_in = (kk == 2 * (jj % P) + jj // P).astype(jnp.bfloat16)
    pm_out = (jj == 2 * (kk % P) + kk // P).astype(jnp.bfloat16)

    tok_d = tok_ref[...]
    bn = tok_d.shape[0]
    tok = _permute_exact(tok_d, pm_in)
    residual = tok
    z_q = jnp.zeros_like(tok)
    iota_k = jax.lax.broadcasted_iota(jnp.int32, (bn, K), 1)
    for r in range(R):
        cb = embed_ref[r]
        cb2 = cb2_ref[r]
        r2 = jnp.sum(residual * residual, axis=1, keepdims=True)
        mm = jax.lax.dot_general(-2.0 * residual, cb, (((1,), (1,)), ((), ())),
                                 preferred_element_type=jnp.float32)
        d = r2 + mm + cb2[None, :]
        dmin = jnp.min(d, axis=1, keepdims=True)
        idx = jnp.min(jnp.where(d == dmin, iota_k, K), axis=1, keepdims=True)
        onehot = (iota_k == idx).astype(jnp.bfloat16)
        q3 = jax.lax.dot_general(onehot, cbpack_ref[r], (((1,), (0,)), ((), ())),
                                 preferred_element_type=jnp.float32)
        q = (q3[:, :D] + q3[:, D:2 * D]) + q3[:, 2 * D:]
        z_q = z_q + q
        residual = residual - q
        codes_ref[:, pl.ds(r, 1)] = idx
        counts_ref[pl.ds(r, 1), :] += jnp.sum(
            onehot.astype(jnp.float32), axis=0, keepdims=True)

    z_q_d = _permute_exact(z_q, pm_out)
    xq_ref[...] = z_q_d
    wexp = wexp_ref[...]
    diff = z_q_d - tok_d
    acc_ref[0] += jnp.sum(diff * diff * wexp)
    acc_ref[1] += jnp.sum(wexp)

    @pl.when(i == nsteps - 1)
    def _fin():
        used = (counts_ref[...] > 0).astype(jnp.float32)
        usage_ref[...] = jnp.mean(used, axis=1, keepdims=True)
        den = jnp.maximum(acc_ref[1] * 0.5, EPS)
        loss_ref[...] = jnp.full((1, 1), acc_ref[0] / den, dtype=jnp.float32)


def kernel(x, w, embed):
    Bx, Lx, Cx = x.shape
    D = P * Cx
    T = Lx // P
    N = Bx * T
    xa = x.reshape(N, P, Cx)
    tok_d = jnp.concatenate([xa[:, :, 0], xa[:, :, 1]], axis=1)
    wr = w.reshape(N, P)
    wexp_d = jnp.concatenate([wr, wr], axis=1)

    BN = 4096
    grid = (N // BN,)

    xq, codes, usage, loss = pl.pallas_call(
        _rvq_body,
        grid=grid,
        in_specs=[
            pl.BlockSpec((BN, D), lambda i: (i, 0)),
            pl.BlockSpec((BN, D), lambda i: (i, 0)),
            pl.BlockSpec((R, K, D), lambda i: (0, 0, 0)),
        ],
        out_specs=[
            pl.BlockSpec((BN, D), lambda i: (i, 0)),
            pl.BlockSpec((BN, R), lambda i: (i, 0)),
            pl.BlockSpec((R, 1), lambda i: (0, 0)),
            pl.BlockSpec((1, 1), lambda i: (0, 0)),
        ],
        out_shape=[
            jax.ShapeDtypeStruct((N, D), jnp.float32),
            jax.ShapeDtypeStruct((N, R), jnp.int32),
            jax.ShapeDtypeStruct((R, 1), jnp.float32),
            jax.ShapeDtypeStruct((1, 1), jnp.float32),
        ],
        scratch_shapes=[
            pltpu.VMEM((R, K), jnp.float32),
            pltpu.SMEM((2,), jnp.float32),
            pltpu.VMEM((R, K), jnp.float32),
            pltpu.VMEM((R, K, 192), jnp.bfloat16),
        ],
        compiler_params=pltpu.CompilerParams(
            dimension_semantics=("arbitrary",)),
    )(tok_d, wexp_d, embed)

    x_q = jnp.stack([xq[:, :P], xq[:, P:]], axis=-1).reshape(Bx, T * P, Cx)
    return loss[0, 0], x_q, codes.reshape(Bx, T, R), usage.reshape(R)


# speedup vs baseline: 1.2235x; 1.2235x over previous
import jax
import jax.numpy as jnp
from jax.experimental import pallas as pl
from jax.experimental.pallas import tpu as pltpu

P = 32
K = 1024
R = 4
EPS = 1e-6


def _split3(v):
    hi = v.astype(jnp.bfloat16)
    rem = v - hi.astype(jnp.float32)
    md = rem.astype(jnp.bfloat16)
    lo = (rem - md.astype(jnp.float32)).astype(jnp.bfloat16)
    return hi, md, lo


def _permute_exact(v, pm):
    hi, md, lo = _split3(v)
    dims = (((1,), (0,)), ((), ()))
    ph = jax.lax.dot_general(hi, pm, dims, preferred_element_type=jnp.float32)
    pmd = jax.lax.dot_general(md, pm, dims, preferred_element_type=jnp.float32)
    plo = jax.lax.dot_general(lo, pm, dims, preferred_element_type=jnp.float32)
    return (ph + pmd) + plo


def _rvq_body(c0_ref, c1_ref, wr_ref, embed_ref,
              xq_ref, codes_ref, usage_ref, loss_ref,
              counts_ref, acc_ref, cb2_ref, cbpack_ref):
    i = pl.program_id(0)
    nsteps = pl.num_programs(0)
    D = 2 * c0_ref.shape[1]

    @pl.when(i == 0)
    def _init():
        counts_ref[...] = jnp.zeros_like(counts_ref)
        acc_ref[0] = 0.0
        acc_ref[1] = 0.0
        emb = embed_ref[...]
        cb2_ref[...] = jnp.sum(emb * emb, axis=2)
        hi, md, lo = _split3(emb)
        cbpack_ref[...] = jnp.concatenate([hi, md, lo], axis=2)

    jj = jax.lax.broadcasted_iota(jnp.int32, (D, D), 0)
    kk = jax.lax.broadcasted_iota(jnp.int32, (D, D), 1)
    pm_in = (kk == 2 * (jj % P) + jj // P).astype(jnp.bfloat16)
    pm_out = (jj == 2 * (kk % P) + kk // P).astype(jnp.bfloat16)

    tok_d = jnp.concatenate([c0_ref[...], c1_ref[...]], axis=1)
    bn = tok_d.shape[0]
    tok = _permute_exact(tok_d, pm_in)
    residual = tok
    z_q = jnp.zeros_like(tok)
    iota_k = jax.lax.broadcasted_iota(jnp.int32, (bn, K), 1)
    for r in range(R):
        cb = embed_ref[r]
        cb2 = cb2_ref[r]
        r2 = jnp.sum(residual * residual, axis=1, keepdims=True)
        mm = jax.lax.dot_general(-2.0 * residual, cb, (((1,), (1,)), ((), ())),
                                 preferred_element_type=jnp.float32)
        d = r2 + mm + cb2[None, :]
        dmin = jnp.min(d, axis=1, keepdims=True)
        idx = jnp.min(jnp.where(d == dmin, iota_k, K), axis=1, keepdims=True)
        onehot = (iota_k == idx).astype(jnp.bfloat16)
        q3 = jax.lax.dot_general(onehot, cbpack_ref[r], (((1,), (0,)), ((), ())),
                                 preferred_element_type=jnp.float32)
        q = (q3[:, :D] + q3[:, D:2 * D]) + q3[:, 2 * D:]
        z_q = z_q + q
        residual = residual - q
        codes_ref[:, pl.ds(r, 1)] = idx
        counts_ref[pl.ds(r, 1), :] += jnp.sum(
            onehot.astype(jnp.float32), axis=0, keepdims=True)

    z_q_d = _permute_exact(z_q, pm_out)
    xq_ref[...] = z_q_d
    wr = wr_ref[...]
    diff = z_q_d - tok_d
    d2 = diff * diff
    acc_ref[0] += jnp.sum((d2[:, :P] + d2[:, P:]) * wr)
    acc_ref[1] += jnp.sum(wr)

    @pl.when(i == nsteps - 1)
    def _fin():
        used = (counts_ref[...] > 0).astype(jnp.float32)
        usage_ref[...] = jnp.mean(used, axis=1, keepdims=True)
        den = jnp.maximum(acc_ref[1], EPS)
        loss_ref[...] = jnp.full((1, 1), acc_ref[0] / den, dtype=jnp.float32)


def kernel(x, w, embed):
    Bx, Lx, Cx = x.shape
    D = P * Cx
    T = Lx // P
    N = Bx * T
    xa = x.reshape(N, P, Cx)
    c0, c1 = xa[:, :, 0], xa[:, :, 1]
    wr = w.reshape(N, P)

    BN = 2048
    grid = (N // BN,)

    xq, codes, usage, loss = pl.pallas_call(
        _rvq_body,
        grid=grid,
        in_specs=[
            pl.BlockSpec((BN, P), lambda i: (i, 0)),
            pl.BlockSpec((BN, P), lambda i: (i, 0)),
            pl.BlockSpec((BN, P), lambda i: (i, 0)),
            pl.BlockSpec((R, K, D), lambda i: (0, 0, 0)),
        ],
        out_specs=[
            pl.BlockSpec((BN, D), lambda i: (i, 0)),
            pl.BlockSpec((BN, R), lambda i: (i, 0)),
            pl.BlockSpec((R, 1), lambda i: (0, 0)),
            pl.BlockSpec((1, 1), lambda i: (0, 0)),
        ],
        out_shape=[
            jax.ShapeDtypeStruct((N, D), jnp.float32),
            jax.ShapeDtypeStruct((N, R), jnp.int32),
            jax.ShapeDtypeStruct((R, 1), jnp.float32),
            jax.ShapeDtypeStruct((1, 1), jnp.float32),
        ],
        scratch_shapes=[
            pltpu.VMEM((R, K), jnp.float32),
            pltpu.SMEM((2,), jnp.float32),
            pltpu.VMEM((R, K), jnp.float32),
            pltpu.VMEM((R, K, 192), jnp.bfloat16),
        ],
        compiler_params=pltpu.CompilerParams(
            dimension_semantics=("arbitrary",)),
    )(c0, c1, wr, embed)

    x_q = jnp.stack([xq[:, :P], xq[:, P:]], axis=-1).reshape(Bx, T * P, Cx)
    return loss[0, 0], x_q, codes.reshape(Bx, T, R), usage.reshape(R)


# speedup vs baseline: 1.2665x; 1.0352x over previous
import jax
import jax.numpy as jnp
from jax.experimental import pallas as pl
from jax.experimental.pallas import tpu as pltpu

P = 32
K = 1024
R = 4
EPS = 1e-6


def _split3(v):
    hi = v.astype(jnp.bfloat16)
    rem = v - hi.astype(jnp.float32)
    md = rem.astype(jnp.bfloat16)
    lo = (rem - md.astype(jnp.float32)).astype(jnp.bfloat16)
    return hi, md, lo


def _permute_exact(v, pm):
    hi, md, lo = _split3(v)
    dims = (((1,), (0,)), ((), ()))
    ph = jax.lax.dot_general(hi, pm, dims, preferred_element_type=jnp.float32)
    pmd = jax.lax.dot_general(md, pm, dims, preferred_element_type=jnp.float32)
    plo = jax.lax.dot_general(lo, pm, dims, preferred_element_type=jnp.float32)
    return (ph + pmd) + plo


def _rvq_body(c0_ref, c1_ref, wr_ref, embed_ref,
              xq0_ref, xq1_ref, codes_ref, usage_ref, loss_ref,
              counts_ref, acc_ref, cb2_ref, cbpack_ref):
    i = pl.program_id(0)
    nsteps = pl.num_programs(0)
    D = 2 * c0_ref.shape[1]

    @pl.when(i == 0)
    def _init():
        counts_ref[...] = jnp.zeros_like(counts_ref)
        acc_ref[0] = 0.0
        acc_ref[1] = 0.0
        emb = embed_ref[...]
        cb2_ref[...] = jnp.sum(emb * emb, axis=2)
        hi, md, lo = _split3(emb)
        cbpack_ref[...] = jnp.concatenate([hi, md, lo], axis=2)

    jj = jax.lax.broadcasted_iota(jnp.int32, (D, D), 0)
    kk = jax.lax.broadcasted_iota(jnp.int32, (D, D), 1)
    pm_in = (kk == 2 * (jj % P) + jj // P).astype(jnp.bfloat16)
    pm_out = (jj == 2 * (kk % P) + kk // P).astype(jnp.bfloat16)

    tok_d = jnp.concatenate([c0_ref[...], c1_ref[...]], axis=1)
    bn = tok_d.shape[0]
    tok = _permute_exact(tok_d, pm_in)
    residual = tok
    z_q = jnp.zeros_like(tok)
    iota_k = jax.lax.broadcasted_iota(jnp.int32, (bn, K), 1)
    for r in range(R):
        cb = embed_ref[r]
        cb2 = cb2_ref[r]
        r2 = jnp.sum(residual * residual, axis=1, keepdims=True)
        mm = jax.lax.dot_general(-2.0 * residual, cb, (((1,), (1,)), ((), ())),
                                 preferred_element_type=jnp.float32)
        d = r2 + mm + cb2[None, :]
        dmin = jnp.min(d, axis=1, keepdims=True)
        idx = jnp.min(jnp.where(d == dmin, iota_k, K), axis=1, keepdims=True)
        onehot = (iota_k == idx).astype(jnp.bfloat16)
        q3 = jax.lax.dot_general(onehot, cbpack_ref[r], (((1,), (0,)), ((), ())),
                                 preferred_element_type=jnp.float32)
        q = (q3[:, :D] + q3[:, D:2 * D]) + q3[:, 2 * D:]
        z_q = z_q + q
        residual = residual - q
        codes_ref[:, pl.ds(r, 1)] = idx
        counts_ref[pl.ds(r, 1), :] += jnp.sum(
            onehot.astype(jnp.float32), axis=0, keepdims=True)

    z_q_d = _permute_exact(z_q, pm_out)
    xq0_ref[...] = z_q_d[:, :P]
    xq1_ref[...] = z_q_d[:, P:]
    wr = wr_ref[...]
    diff = z_q_d - tok_d
    d2 = diff * diff
    acc_ref[0] += jnp.sum((d2[:, :P] + d2[:, P:]) * wr)
    acc_ref[1] += jnp.sum(wr)

    @pl.when(i == nsteps - 1)
    def _fin():
        used = (counts_ref[...] > 0).astype(jnp.float32)
        usage_ref[...] = jnp.mean(used, axis=1, keepdims=True)
        den = jnp.maximum(acc_ref[1], EPS)
        loss_ref[...] = jnp.full((1, 1), acc_ref[0] / den, dtype=jnp.float32)


def kernel(x, w, embed):
    Bx, Lx, Cx = x.shape
    D = P * Cx
    T = Lx // P
    N = Bx * T
    xa = x.reshape(N, P, Cx)
    c0, c1 = xa[:, :, 0], xa[:, :, 1]
    wr = w.reshape(N, P)

    BN = 2048
    grid = (N // BN,)

    xq0, xq1, codes, usage, loss = pl.pallas_call(
        _rvq_body,
        grid=grid,
        in_specs=[
            pl.BlockSpec((BN, P), lambda i: (i, 0)),
            pl.BlockSpec((BN, P), lambda i: (i, 0)),
            pl.BlockSpec((BN, P), lambda i: (i, 0)),
            pl.BlockSpec((R, K, D), lambda i: (0, 0, 0)),
        ],
        out_specs=[
            pl.BlockSpec((BN, P), lambda i: (i, 0)),
            pl.BlockSpec((BN, P), lambda i: (i, 0)),
            pl.BlockSpec((BN, R), lambda i: (i, 0)),
            pl.BlockSpec((R, 1), lambda i: (0, 0)),
            pl.BlockSpec((1, 1), lambda i: (0, 0)),
        ],
        out_shape=[
            jax.ShapeDtypeStruct((N, P), jnp.float32),
            jax.ShapeDtypeStruct((N, P), jnp.float32),
            jax.ShapeDtypeStruct((N, R), jnp.int32),
            jax.ShapeDtypeStruct((R, 1), jnp.float32),
            jax.ShapeDtypeStruct((1, 1), jnp.float32),
        ],
        scratch_shapes=[
            pltpu.VMEM((R, K), jnp.float32),
            pltpu.SMEM((2,), jnp.float32),
            pltpu.VMEM((R, K), jnp.float32),
            pltpu.VMEM((R, K, 192), jnp.bfloat16),
        ],
        compiler_params=pltpu.CompilerParams(
            dimension_semantics=("arbitrary",)),
    )(c0, c1, wr, embed)

    x_q = jnp.stack([xq0, xq1], axis=-1).reshape(Bx, T * P, Cx)
    return loss[0, 0], x_q, codes.reshape(Bx, T, R), usage.reshape(R)
